# Initial kernel scaffold; baseline (speedup 1.0000x reference)
import jax
import jax.numpy as jnp
from jax.experimental import pallas as pl


def kernel(x, edge_index, edge_weight, W1, b1, W2, b2):
    raise NotImplementedError("write your pallas kernel here")


# speedup vs baseline: 8.2542x; 8.2542x over previous
import functools

import jax
import jax.numpy as jnp
from jax import lax
from jax.experimental import pallas as pl
from jax.experimental.pallas import tpu as pltpu
from jax.experimental.pallas import tpu_sc as plsc

_N = 10000
_E = 320000
_C = 128
_NC = 2
_NS = 16
_NW = _NC * _NS
_EPT = _E // _NW
_EB = 80
_NB = _EPT // _EB
_RPT = _N // _NS
_DW = 16
_BLK = 2000

_mesh = plsc.VectorSubcoreMesh(core_axis_name="c", subcore_axis_name="s")


def _zero_rows(ref, nrows, ncols):
    zeros16 = jnp.zeros((16,), jnp.float32)

    def body(i, _):
        for g in range(ncols // 16):
            ref[i, pl.ds(g * 16, 16)] = zeros16
        return 0

    lax.fori_loop(0, nrows, body, 0)


def _over_my_rows(sid, fn):
    base = sid * 624
    for k in range(7):
        fn(base + k * 80, 80)

    @pl.when(sid == _NS - 1)
    def _():
        fn(base + 560, 80)

    @pl.when(sid != _NS - 1)
    def _():
        fn(base + 560, 64)


@functools.partial(
    pl.kernel,
    out_type=jax.ShapeDtypeStruct((_NC * _N,), jnp.float32),
    mesh=_mesh,
    scratch_types=[
        pltpu.VMEM((_EB,), jnp.int32),
        pltpu.VMEM((_EB,), jnp.float32),
        pltpu.VMEM((640,), jnp.float32),
        pltpu.VMEM_SHARED((_N,), jnp.float32),
    ],
)
def _deg_kernel(col_hbm, w_hbm, deg_hbm, col_v, w_v, zbuf, deg_sh):
    cid = lax.axis_index("c")
    sid = lax.axis_index("s")
    wid = cid * _NS + sid
    ebase = wid * _EPT

    zeros16 = jnp.zeros((16,), jnp.float32)

    def zb(i, _):
        zbuf[pl.ds(i * 16, 16)] = zeros16
        return 0

    lax.fori_loop(0, 40, zb, 0)
    base = sid * 624

    @pl.when(sid == _NS - 1)
    def _():
        pltpu.sync_copy(zbuf, deg_sh.at[pl.ds(base, 640)])

    @pl.when(sid != _NS - 1)
    def _():
        pltpu.sync_copy(zbuf.at[pl.ds(0, 624)], deg_sh.at[pl.ds(base, 624)])

    plsc.subcore_barrier()

    def batch(b, _):
        e0 = ebase + b * _EB
        pltpu.sync_copy(col_hbm.at[pl.ds(e0, _EB)], col_v)
        pltpu.sync_copy(w_hbm.at[pl.ds(e0, _EB)], w_v)
        pltpu.sync_copy(w_v, deg_sh.at[col_v], add=True)
        return 0

    lax.fori_loop(0, _NB, batch, 0)
    plsc.subcore_barrier()

    @pl.when(sid == _NS - 1)
    def _():
        pltpu.sync_copy(deg_sh.at[pl.ds(base, 640)], zbuf)
        pltpu.sync_copy(zbuf, deg_hbm.at[pl.ds(cid * _N + base, 640)])

    @pl.when(sid != _NS - 1)
    def _():
        pltpu.sync_copy(deg_sh.at[pl.ds(base, 624)], zbuf.at[pl.ds(0, 624)])
        pltpu.sync_copy(zbuf.at[pl.ds(0, 624)],
                        deg_hbm.at[pl.ds(cid * _N + base, 624)])


@functools.partial(
    pl.kernel,
    out_type=jax.ShapeDtypeStruct((_NC * _N, _C), jnp.float32),
    mesh=_mesh,
    scratch_types=[
        pltpu.VMEM((_EB,), jnp.int32),
        pltpu.VMEM((_EB,), jnp.int32),
        pltpu.VMEM((_EB,), jnp.float32),
        pltpu.VMEM((_EB, _C), jnp.float32),
        pltpu.VMEM_SHARED((_N, _C), jnp.float32),
        pltpu.SemaphoreType.DMA,
    ],
)
def _agg_kernel(h_hbm, row_hbm, col_hbm, w_hbm, out_hbm,
                row_v, col_v, w_v, rows_v, acc, sem):
    cid = lax.axis_index("c")
    sid = lax.axis_index("s")
    wid = cid * _NS + sid
    ebase = wid * _EPT

    _zero_rows(rows_v, _EB, _C)
    _over_my_rows(sid, lambda s, n: pltpu.sync_copy(
        rows_v.at[pl.ds(0, n)], acc.at[pl.ds(s, n)]))
    plsc.subcore_barrier()

    def batch(b, _):
        base = ebase + b * _EB
        pltpu.sync_copy(row_hbm.at[pl.ds(base, _EB)], row_v)
        pltpu.sync_copy(col_hbm.at[pl.ds(base, _EB)], col_v)
        pltpu.sync_copy(w_hbm.at[pl.ds(base, _EB)], w_v)
        pltpu.async_copy(h_hbm.at[row_v], rows_v, sem).wait()

        def egroup(g, _):
            wv16 = w_v[pl.ds(g * 16, 16)]
            for j in range(16):
                e = g * 16 + j
                s = wv16[j]
                for c in range(_C // 16):
                    sl = pl.ds(c * 16, 16)
                    rows_v[e, sl] = rows_v[e, sl] * s
            return 0

        lax.fori_loop(0, _EB // 16, egroup, 0)
        pltpu.sync_copy(rows_v, acc.at[col_v], add=True)
        return 0

    lax.fori_loop(0, _NB, batch, 0)
    plsc.subcore_barrier()

    def _out_chunk(s, n):
        pltpu.sync_copy(acc.at[pl.ds(s, n)], rows_v.at[pl.ds(0, n)])
        pltpu.sync_copy(rows_v.at[pl.ds(0, n)],
                        out_hbm.at[pl.ds(cid * _N + s, n)])

    _over_my_rows(sid, _out_chunk)


def _dinv_of(deg2_ref):
    d = deg2_ref[...]
    deg = d[:, 0] + d[:, 1]
    pos = deg > 0
    return jnp.where(pos, lax.rsqrt(jnp.where(pos, deg, 1.0)), 0.0)


def _lin1_body(deg2_ref, x_ref, w1_ref, h_ref):
    dinv = _dinv_of(deg2_ref)
    h = jnp.dot(x_ref[...], w1_ref[...], preferred_element_type=jnp.float32)
    h_ref[...] = h * dinv[:, None]


def _mid_body(p_ref, deg2_ref, b1_ref, w2_ref, h2_ref):
    dinv = _dinv_of(deg2_ref)
    out1 = (p_ref[0] + p_ref[1]) * dinv[:, None] + b1_ref[...]
    h2 = jnp.maximum(out1, 0.0)
    h2_ref[...] = jnp.dot(
        h2, w2_ref[...], preferred_element_type=jnp.float32) * dinv[:, None]


def _fin_body(q_ref, deg2_ref, b2_ref, o_ref):
    dinv = _dinv_of(deg2_ref)
    o_ref[...] = (q_ref[0] + q_ref[1]) * dinv[:, None] + b2_ref[...]


def kernel(x, edge_index, edge_weight, W1, b1, W2, b2):
    row = edge_index[0].astype(jnp.int32)
    col = edge_index[1].astype(jnp.int32)
    w = edge_weight.astype(jnp.float32)

    deg_raw = _deg_kernel(col, w)
    deg2 = deg_raw.reshape(_NC, _N).T
    b1r = b1.reshape(1, _C)
    b2r = b2.reshape(1, _C)
    grid = (_N // _BLK,)

    h1 = pl.pallas_call(
        _lin1_body,
        grid=grid,
        in_specs=[
            pl.BlockSpec((_BLK, 2), lambda i: (i, 0)),
            pl.BlockSpec((_BLK, _C), lambda i: (i, 0)),
            pl.BlockSpec((_C, _C), lambda i: (0, 0)),
        ],
        out_specs=pl.BlockSpec((_BLK, _C), lambda i: (i, 0)),
        out_shape=jax.ShapeDtypeStruct((_N, _C), jnp.float32),
    )(deg2, x, W1)

    p = _agg_kernel(h1, row, col, w).reshape(_NC, _N, _C)

    h2 = pl.pallas_call(
        _mid_body,
        grid=grid,
        in_specs=[
            pl.BlockSpec((2, _BLK, _C), lambda i: (0, i, 0)),
            pl.BlockSpec((_BLK, 2), lambda i: (i, 0)),
            pl.BlockSpec((1, _C), lambda i: (0, 0)),
            pl.BlockSpec((_C, _C), lambda i: (0, 0)),
        ],
        out_specs=pl.BlockSpec((_BLK, _C), lambda i: (i, 0)),
        out_shape=jax.ShapeDtypeStruct((_N, _C), jnp.float32),
    )(p, deg2, b1r, W2)

    q = _agg_kernel(h2, row, col, w).reshape(_NC, _N, _C)

    out = pl.pallas_call(
        _fin_body,
        grid=grid,
        in_specs=[
            pl.BlockSpec((2, _BLK, _C), lambda i: (0, i, 0)),
            pl.BlockSpec((_BLK, 2), lambda i: (i, 0)),
            pl.BlockSpec((1, _C), lambda i: (0, 0)),
        ],
        out_specs=pl.BlockSpec((_BLK, _C), lambda i: (i, 0)),
        out_shape=jax.ShapeDtypeStruct((_N, _C), jnp.float32),
    )(q, deg2, b2r)
    return out


# speedup vs baseline: 18.1157x; 2.1947x over previous
import functools

import jax
import jax.numpy as jnp
from jax import lax
from jax.experimental import pallas as pl
from jax.experimental.pallas import tpu as pltpu
from jax.experimental.pallas import tpu_sc as plsc

_N = 10000
_E = 320000
_C = 128
_NC = 2
_NS = 16
_NW = _NC * _NS
_EPT = _E // _NW
_EB = 80
_NB = _EPT // _EB
_NP = (_NB - 1) // 2
_BLK = 2000

_mesh = plsc.VectorSubcoreMesh(core_axis_name="c", subcore_axis_name="s")


def _zero_rows(ref, nrows, ncols):
    zeros16 = jnp.zeros((16,), jnp.float32)

    def body(i, _):
        for g in range(ncols // 16):
            ref[i, pl.ds(g * 16, 16)] = zeros16
        return 0

    lax.fori_loop(0, nrows, body, 0)


def _over_my_rows(sid, fn):
    base = sid * 624
    for k in range(7):
        fn(base + k * 80, 80)

    @pl.when(sid == _NS - 1)
    def _():
        fn(base + 560, 80)

    @pl.when(sid != _NS - 1)
    def _():
        fn(base + 560, 64)


@functools.partial(
    pl.kernel,
    out_type=jax.ShapeDtypeStruct((_NC * _N,), jnp.float32),
    mesh=_mesh,
    scratch_types=[
        pltpu.VMEM((_EB,), jnp.int32),
        pltpu.VMEM((_EB,), jnp.float32),
        pltpu.VMEM((_EB,), jnp.int32),
        pltpu.VMEM((_EB,), jnp.float32),
        pltpu.VMEM((640,), jnp.float32),
        pltpu.VMEM_SHARED((_N,), jnp.float32),
        pltpu.SemaphoreType.DMA,
        pltpu.SemaphoreType.DMA,
    ],
)
def _deg_kernel(col_hbm, w_hbm, deg_hbm,
                col0, w0, col1, w1, zbuf, deg_sh, si0, si1):
    cid = lax.axis_index("c")
    sid = lax.axis_index("s")
    wid = cid * _NS + sid
    ebase = wid * _EPT

    zeros16 = jnp.zeros((16,), jnp.float32)

    def zb(i, _):
        zbuf[pl.ds(i * 16, 16)] = zeros16
        return 0

    lax.fori_loop(0, 40, zb, 0)
    base = sid * 624

    @pl.when(sid == _NS - 1)
    def _():
        pltpu.sync_copy(zbuf, deg_sh.at[pl.ds(base, 640)])

    @pl.when(sid != _NS - 1)
    def _():
        pltpu.sync_copy(zbuf.at[pl.ds(0, 624)], deg_sh.at[pl.ds(base, 624)])

    plsc.subcore_barrier()

    pltpu.sync_copy(col_hbm.at[pl.ds(ebase, _EB)], col0)
    pltpu.sync_copy(w_hbm.at[pl.ds(ebase, _EB)], w0)
    pltpu.async_copy(col_hbm.at[pl.ds(ebase + _EB, _EB)], col1, si1)
    pltpu.async_copy(w_hbm.at[pl.ds(ebase + _EB, _EB)], w1, si1)

    def pair(i, _):
        b0 = 2 * i
        e1 = ebase + (b0 + 1) * _EB
        e2 = ebase + (b0 + 2) * _EB
        e3 = ebase + (b0 + 3) * _EB
        pltpu.sync_copy(w0, deg_sh.at[col0], add=True)
        pltpu.async_copy(col_hbm.at[pl.ds(e2, _EB)], col0, si0)
        pltpu.async_copy(w_hbm.at[pl.ds(e2, _EB)], w0, si0)
        pltpu.make_async_copy(col_hbm.at[pl.ds(e1, _EB)], col1, si1).wait()
        pltpu.make_async_copy(w_hbm.at[pl.ds(e1, _EB)], w1, si1).wait()
        pltpu.sync_copy(w1, deg_sh.at[col1], add=True)

        @pl.when(b0 + 3 < _NB)
        def _():
            pltpu.async_copy(col_hbm.at[pl.ds(e3, _EB)], col1, si1)
            pltpu.async_copy(w_hbm.at[pl.ds(e3, _EB)], w1, si1)

        pltpu.make_async_copy(col_hbm.at[pl.ds(e2, _EB)], col0, si0).wait()
        pltpu.make_async_copy(w_hbm.at[pl.ds(e2, _EB)], w0, si0).wait()
        return 0

    lax.fori_loop(0, _NP, pair, 0)
    pltpu.sync_copy(w0, deg_sh.at[col0], add=True)
    plsc.subcore_barrier()

    @pl.when(sid == _NS - 1)
    def _():
        pltpu.sync_copy(deg_sh.at[pl.ds(base, 640)], zbuf)
        pltpu.sync_copy(zbuf, deg_hbm.at[pl.ds(cid * _N + base, 640)])

    @pl.when(sid != _NS - 1)
    def _():
        pltpu.sync_copy(deg_sh.at[pl.ds(base, 624)], zbuf.at[pl.ds(0, 624)])
        pltpu.sync_copy(zbuf.at[pl.ds(0, 624)],
                        deg_hbm.at[pl.ds(cid * _N + base, 624)])


def _scale_rows(rows_v, w_v):
    def egroup(g, _):
        wv16 = w_v[pl.ds(g * 16, 16)]
        for t in range(16):
            e = g * 16 + t
            s = wv16[t]
            for c in range(_C // 16):
                sl = pl.ds(c * 16, 16)
                rows_v[e, sl] = rows_v[e, sl] * s
        return 0

    lax.fori_loop(0, _EB // 16, egroup, 0)


@functools.partial(
    pl.kernel,
    out_type=jax.ShapeDtypeStruct((_NC * _N, _C), jnp.float32),
    mesh=_mesh,
    scratch_types=[
        pltpu.VMEM((_EB,), jnp.int32),
        pltpu.VMEM((_EB,), jnp.int32),
        pltpu.VMEM((_EB,), jnp.float32),
        pltpu.VMEM((_EB,), jnp.int32),
        pltpu.VMEM((_EB,), jnp.int32),
        pltpu.VMEM((_EB,), jnp.float32),
        pltpu.VMEM((_EB, _C), jnp.float32),
        pltpu.VMEM((_EB, _C), jnp.float32),
        pltpu.VMEM_SHARED((_N, _C), jnp.float32),
        pltpu.SemaphoreType.DMA,
        pltpu.SemaphoreType.DMA,
        pltpu.SemaphoreType.DMA,
        pltpu.SemaphoreType.DMA,
    ],
)
def _agg_kernel(h_hbm, row_hbm, col_hbm, w_hbm, out_hbm,
                row0, col0, w0, row1, col1, w1, rows0, rows1,
                acc, sg0, sg1, si0, si1):
    cid = lax.axis_index("c")
    sid = lax.axis_index("s")
    wid = cid * _NS + sid
    ebase = wid * _EPT

    _zero_rows(rows0, _EB, _C)
    _over_my_rows(sid, lambda s, n: pltpu.sync_copy(
        rows0.at[pl.ds(0, n)], acc.at[pl.ds(s, n)]))
    plsc.subcore_barrier()

    pltpu.sync_copy(row_hbm.at[pl.ds(ebase, _EB)], row0)
    pltpu.sync_copy(col_hbm.at[pl.ds(ebase, _EB)], col0)
    pltpu.sync_copy(w_hbm.at[pl.ds(ebase, _EB)], w0)
    pltpu.async_copy(h_hbm.at[row0], rows0, sg0)
    pltpu.async_copy(row_hbm.at[pl.ds(ebase + _EB, _EB)], row1, si1)
    pltpu.async_copy(col_hbm.at[pl.ds(ebase + _EB, _EB)], col1, si1)
    pltpu.async_copy(w_hbm.at[pl.ds(ebase + _EB, _EB)], w1, si1)

    def pair(i, _):
        b0 = 2 * i
        e1 = ebase + (b0 + 1) * _EB
        e2 = ebase + (b0 + 2) * _EB
        e3 = ebase + (b0 + 3) * _EB

        pltpu.make_async_copy(row_hbm.at[pl.ds(e1, _EB)], row1, si1).wait()
        pltpu.make_async_copy(col_hbm.at[pl.ds(e1, _EB)], col1, si1).wait()
        pltpu.make_async_copy(w_hbm.at[pl.ds(e1, _EB)], w1, si1).wait()
        pltpu.async_copy(h_hbm.at[row1], rows1, sg1)
        pltpu.make_async_copy(h_hbm.at[row0], rows0, sg0).wait()
        pltpu.async_copy(row_hbm.at[pl.ds(e2, _EB)], row0, si0)
        _scale_rows(rows0, w0)
        pltpu.sync_copy(rows0, acc.at[col0], add=True)
        pltpu.async_copy(col_hbm.at[pl.ds(e2, _EB)], col0, si0)
        pltpu.async_copy(w_hbm.at[pl.ds(e2, _EB)], w0, si0)

        pltpu.make_async_copy(row_hbm.at[pl.ds(e2, _EB)], row0, si0).wait()
        pltpu.make_async_copy(col_hbm.at[pl.ds(e2, _EB)], col0, si0).wait()
        pltpu.make_async_copy(w_hbm.at[pl.ds(e2, _EB)], w0, si0).wait()
        pltpu.async_copy(h_hbm.at[row0], rows0, sg0)
        pltpu.make_async_copy(h_hbm.at[row1], rows1, sg1).wait()

        @pl.when(b0 + 3 < _NB)
        def _():
            pltpu.async_copy(row_hbm.at[pl.ds(e3, _EB)], row1, si1)

        _scale_rows(rows1, w1)
        pltpu.sync_copy(rows1, acc.at[col1], add=True)

        @pl.when(b0 + 3 < _NB)
        def _():
            pltpu.async_copy(col_hbm.at[pl.ds(e3, _EB)], col1, si1)
            pltpu.async_copy(w_hbm.at[pl.ds(e3, _EB)], w1, si1)

        return 0

    lax.fori_loop(0, _NP, pair, 0)
    pltpu.make_async_copy(h_hbm.at[row0], rows0, sg0).wait()
    _scale_rows(rows0, w0)
    pltpu.sync_copy(rows0, acc.at[col0], add=True)
    plsc.subcore_barrier()

    def _out_chunk(s, n):
        pltpu.sync_copy(acc.at[pl.ds(s, n)], rows0.at[pl.ds(0, n)])
        pltpu.sync_copy(rows0.at[pl.ds(0, n)],
                        out_hbm.at[pl.ds(cid * _N + s, n)])

    _over_my_rows(sid, _out_chunk)


def _dinv_of(deg2_ref):
    d = deg2_ref[...]
    deg = d[:, 0] + d[:, 1]
    pos = deg > 0
    return jnp.where(pos, lax.rsqrt(jnp.where(pos, deg, 1.0)), 0.0)


def _lin1_body(deg2_ref, x_ref, w1_ref, h_ref):
    dinv = _dinv_of(deg2_ref)
    h = jnp.dot(x_ref[...], w1_ref[...], preferred_element_type=jnp.float32)
    h_ref[...] = h * dinv[:, None]


def _mid_body(p_ref, deg2_ref, b1_ref, w2_ref, h2_ref):
    dinv = _dinv_of(deg2_ref)
    out1 = (p_ref[0] + p_ref[1]) * dinv[:, None] + b1_ref[...]
    h2 = jnp.maximum(out1, 0.0)
    h2_ref[...] = jnp.dot(
        h2, w2_ref[...], preferred_element_type=jnp.float32) * dinv[:, None]


def _fin_body(q_ref, deg2_ref, b2_ref, o_ref):
    dinv = _dinv_of(deg2_ref)
    o_ref[...] = (q_ref[0] + q_ref[1]) * dinv[:, None] + b2_ref[...]


def kernel(x, edge_index, edge_weight, W1, b1, W2, b2):
    row = edge_index[0].astype(jnp.int32)
    col = edge_index[1].astype(jnp.int32)
    w = edge_weight.astype(jnp.float32)

    deg_raw = _deg_kernel(col, w)
    deg2 = deg_raw.reshape(_NC, _N).T
    b1r = b1.reshape(1, _C)
    b2r = b2.reshape(1, _C)
    grid = (_N // _BLK,)

    h1 = pl.pallas_call(
        _lin1_body,
        grid=grid,
        in_specs=[
            pl.BlockSpec((_BLK, 2), lambda i: (i, 0)),
            pl.BlockSpec((_BLK, _C), lambda i: (i, 0)),
            pl.BlockSpec((_C, _C), lambda i: (0, 0)),
        ],
        out_specs=pl.BlockSpec((_BLK, _C), lambda i: (i, 0)),
        out_shape=jax.ShapeDtypeStruct((_N, _C), jnp.float32),
    )(deg2, x, W1)

    p = _agg_kernel(h1, row, col, w).reshape(_NC, _N, _C)

    h2 = pl.pallas_call(
        _mid_body,
        grid=grid,
        in_specs=[
            pl.BlockSpec((2, _BLK, _C), lambda i: (0, i, 0)),
            pl.BlockSpec((_BLK, 2), lambda i: (i, 0)),
            pl.BlockSpec((1, _C), lambda i: (0, 0)),
            pl.BlockSpec((_C, _C), lambda i: (0, 0)),
        ],
        out_specs=pl.BlockSpec((_BLK, _C), lambda i: (i, 0)),
        out_shape=jax.ShapeDtypeStruct((_N, _C), jnp.float32),
    )(p, deg2, b1r, W2)

    q = _agg_kernel(h2, row, col, w).reshape(_NC, _N, _C)

    out = pl.pallas_call(
        _fin_body,
        grid=grid,
        in_specs=[
            pl.BlockSpec((2, _BLK, _C), lambda i: (0, i, 0)),
            pl.BlockSpec((_BLK, 2), lambda i: (i, 0)),
            pl.BlockSpec((1, _C), lambda i: (0, 0)),
        ],
        out_specs=pl.BlockSpec((_BLK, _C), lambda i: (i, 0)),
        out_shape=jax.ShapeDtypeStruct((_N, _C), jnp.float32),
    )(q, deg2, b2r)
    return out


# speedup vs baseline: 20.2368x; 1.1171x over previous
import functools

import jax
import jax.numpy as jnp
from jax import lax
from jax.experimental import pallas as pl
from jax.experimental.pallas import tpu as pltpu
from jax.experimental.pallas import tpu_sc as plsc

_N = 10000
_E = 320000
_C = 128
_NC = 2
_NS = 16
_NW = _NC * _NS
_EPT = _E // _NW
_EB = 80
_NB = _EPT // _EB
_NP = (_NB - 1) // 2
_BLK = 2000

_mesh = plsc.VectorSubcoreMesh(core_axis_name="c", subcore_axis_name="s")


def _zero_rows(ref, nrows, ncols):
    zeros16 = jnp.zeros((16,), jnp.float32)

    def body(i, _):
        for g in range(ncols // 16):
            ref[i, pl.ds(g * 16, 16)] = zeros16
        return 0

    lax.fori_loop(0, nrows, body, 0)


def _over_my_rows(sid, fn):
    base = sid * 624
    for k in range(7):
        fn(base + k * 80, 80)

    @pl.when(sid == _NS - 1)
    def _():
        fn(base + 560, 80)

    @pl.when(sid != _NS - 1)
    def _():
        fn(base + 560, 64)


@functools.partial(
    pl.kernel,
    out_type=jax.ShapeDtypeStruct((_NC * _N,), jnp.float32),
    mesh=_mesh,
    scratch_types=[
        pltpu.VMEM((_EB,), jnp.int32),
        pltpu.VMEM((_EB,), jnp.float32),
        pltpu.VMEM((_EB,), jnp.int32),
        pltpu.VMEM((_EB,), jnp.float32),
        pltpu.VMEM((640,), jnp.float32),
        pltpu.VMEM_SHARED((_N,), jnp.float32),
        pltpu.SemaphoreType.DMA,
        pltpu.SemaphoreType.DMA,
    ],
)
def _deg_kernel(col_hbm, w_hbm, deg_hbm,
                col0, w0, col1, w1, zbuf, deg_sh, si0, si1):
    cid = lax.axis_index("c")
    sid = lax.axis_index("s")
    wid = cid * _NS + sid
    ebase = wid * _EPT

    zeros16 = jnp.zeros((16,), jnp.float32)

    def zb(i, _):
        zbuf[pl.ds(i * 16, 16)] = zeros16
        return 0

    lax.fori_loop(0, 40, zb, 0)
    base = sid * 624

    @pl.when(sid == _NS - 1)
    def _():
        pltpu.sync_copy(zbuf, deg_sh.at[pl.ds(base, 640)])

    @pl.when(sid != _NS - 1)
    def _():
        pltpu.sync_copy(zbuf.at[pl.ds(0, 624)], deg_sh.at[pl.ds(base, 624)])

    plsc.subcore_barrier()

    pltpu.sync_copy(col_hbm.at[pl.ds(ebase, _EB)], col0)
    pltpu.sync_copy(w_hbm.at[pl.ds(ebase, _EB)], w0)
    pltpu.async_copy(col_hbm.at[pl.ds(ebase + _EB, _EB)], col1, si1)
    pltpu.async_copy(w_hbm.at[pl.ds(ebase + _EB, _EB)], w1, si1)

    def pair(i, _):
        b0 = 2 * i
        e1 = ebase + (b0 + 1) * _EB
        e2 = ebase + (b0 + 2) * _EB
        e3 = ebase + (b0 + 3) * _EB
        pltpu.sync_copy(w0, deg_sh.at[col0], add=True)
        pltpu.async_copy(col_hbm.at[pl.ds(e2, _EB)], col0, si0)
        pltpu.async_copy(w_hbm.at[pl.ds(e2, _EB)], w0, si0)
        pltpu.make_async_copy(col_hbm.at[pl.ds(e1, _EB)], col1, si1).wait()
        pltpu.make_async_copy(w_hbm.at[pl.ds(e1, _EB)], w1, si1).wait()
        pltpu.sync_copy(w1, deg_sh.at[col1], add=True)

        @pl.when(b0 + 3 < _NB)
        def _():
            pltpu.async_copy(col_hbm.at[pl.ds(e3, _EB)], col1, si1)
            pltpu.async_copy(w_hbm.at[pl.ds(e3, _EB)], w1, si1)

        pltpu.make_async_copy(col_hbm.at[pl.ds(e2, _EB)], col0, si0).wait()
        pltpu.make_async_copy(w_hbm.at[pl.ds(e2, _EB)], w0, si0).wait()
        return 0

    lax.fori_loop(0, _NP, pair, 0)
    pltpu.sync_copy(w0, deg_sh.at[col0], add=True)
    plsc.subcore_barrier()

    @pl.when(sid == _NS - 1)
    def _():
        pltpu.sync_copy(deg_sh.at[pl.ds(base, 640)], zbuf)
        pltpu.sync_copy(zbuf, deg_hbm.at[pl.ds(cid * _N + base, 640)])

    @pl.when(sid != _NS - 1)
    def _():
        pltpu.sync_copy(deg_sh.at[pl.ds(base, 624)], zbuf.at[pl.ds(0, 624)])
        pltpu.sync_copy(zbuf.at[pl.ds(0, 624)],
                        deg_hbm.at[pl.ds(cid * _N + base, 624)])


def _scale_rows(rows_v, w_v):
    def egroup(g, _):
        wv16 = w_v[pl.ds(g * 16, 16)]
        for t in range(16):
            e = g * 16 + t
            s = wv16[t]
            for c in range(_C // 16):
                sl = pl.ds(c * 16, 16)
                rows_v[e, sl] = rows_v[e, sl] * s
        return 0

    lax.fori_loop(0, _EB // 16, egroup, 0)


@functools.partial(
    pl.kernel,
    out_type=jax.ShapeDtypeStruct((_NC * _N, _C), jnp.float32),
    mesh=_mesh,
    scratch_types=[
        pltpu.VMEM((_EB,), jnp.int32),
        pltpu.VMEM((_EB,), jnp.int32),
        pltpu.VMEM((_EB,), jnp.float32),
        pltpu.VMEM((_EB,), jnp.int32),
        pltpu.VMEM((_EB,), jnp.int32),
        pltpu.VMEM((_EB,), jnp.float32),
        pltpu.VMEM((_EB,), jnp.int32),
        pltpu.VMEM((_EB,), jnp.int32),
        pltpu.VMEM((_EB, _C), jnp.float32),
        pltpu.VMEM((_EB, _C), jnp.float32),
        pltpu.VMEM((_EB, _C), jnp.float32),
        pltpu.VMEM_SHARED((_N, _C), jnp.float32),
        pltpu.SemaphoreType.DMA,
        pltpu.SemaphoreType.DMA,
        pltpu.SemaphoreType.DMA,
        pltpu.SemaphoreType.DMA,
        pltpu.SemaphoreType.DMA,
        pltpu.SemaphoreType.DMA,
        pltpu.SemaphoreType.DMA,
        pltpu.SemaphoreType.DMA,
    ],
)
def _agg_kernel(h_hbm, row_hbm, col_hbm, w_hbm, out_hbm,
                row0, col0, w0, row1, col1, w1, cs0, cs1,
                ra, rb, rc, acc,
                sg0, sg1, sg2, ss0, ss1, ss2, si0, si1):
    cid = lax.axis_index("c")
    sid = lax.axis_index("s")
    wid = cid * _NS + sid
    ebase = wid * _EPT

    rows = (ra, rb, rc)
    sgs = (sg0, sg1, sg2)
    sss = (ss0, ss1, ss2)
    idx = ((row0, col0, w0, cs0, si0), (row1, col1, w1, cs1, si1))

    _zero_rows(ra, _EB, _C)
    _over_my_rows(sid, lambda s, n: pltpu.sync_copy(
        ra.at[pl.ds(0, n)], acc.at[pl.ds(s, n)]))
    plsc.subcore_barrier()

    def half(b, r3, k, drain_prev, gather_next, prefetch, sync_scatter):
        rB, sgB, ssB = rows[r3], sgs[r3], sss[r3]
        r3n = (r3 + 1) % 3
        rN, sgN, ssP = rows[r3n], sgs[r3n], sss[r3n]
        rowK, colK, wK, csK, siK = idx[k]
        rowN, colN, wN, csN, siN = idx[1 - k]
        e1 = ebase + (b + 1) * _EB
        e2 = ebase + (b + 2) * _EB
        if drain_prev:
            pltpu.make_async_copy(rN, acc.at[csK], ssP).wait()
        if gather_next:
            pltpu.make_async_copy(
                row_hbm.at[pl.ds(e1, _EB)], rowN, siN).wait()
            pltpu.make_async_copy(
                col_hbm.at[pl.ds(e1, _EB)], colN, siN).wait()
            pltpu.make_async_copy(w_hbm.at[pl.ds(e1, _EB)], wN, siN).wait()
            pltpu.async_copy(h_hbm.at[rowN], rN, sgN)
        pltpu.make_async_copy(h_hbm.at[rowK], rB, sgB).wait()
        if prefetch:
            pltpu.async_copy(row_hbm.at[pl.ds(e2, _EB)], rowK, siK)
        _scale_rows(rB, wK)
        if prefetch:
            pltpu.async_copy(w_hbm.at[pl.ds(e2, _EB)], wK, siK)
        if sync_scatter:
            pltpu.sync_copy(rB, acc.at[colK], add=True)
        else:
            for g in range(_EB // 16):
                gs = pl.ds(g * 16, 16)
                csK[gs] = colK[gs]
            pltpu.async_copy(rB, acc.at[csK], ssB, add=True)
        if prefetch:
            pltpu.async_copy(col_hbm.at[pl.ds(e2, _EB)], colK, siK)

    pltpu.sync_copy(row_hbm.at[pl.ds(ebase, _EB)], row0)
    pltpu.sync_copy(col_hbm.at[pl.ds(ebase, _EB)], col0)
    pltpu.sync_copy(w_hbm.at[pl.ds(ebase, _EB)], w0)
    pltpu.async_copy(h_hbm.at[row0], ra, sg0)
    pltpu.async_copy(row_hbm.at[pl.ds(ebase + _EB, _EB)], row1, si1)
    pltpu.async_copy(col_hbm.at[pl.ds(ebase + _EB, _EB)], col1, si1)
    pltpu.async_copy(w_hbm.at[pl.ds(ebase + _EB, _EB)], w1, si1)

    half(0, 0, 0, False, True, True, False)
    half(1, 1, 1, False, True, True, False)

    def six(i, _):
        b0 = 2 + 6 * i
        for u in range(6):
            half(b0 + u, (2 + u) % 3, u % 2, True, True, True, False)
        return 0

    lax.fori_loop(0, (_NB - 5) // 6, six, 0)
    half(_NB - 3, 2, 0, True, True, True, False)
    half(_NB - 2, 0, 1, True, True, False, False)
    half(_NB - 1, 1, 0, True, False, False, True)
    pltpu.make_async_copy(ra, acc.at[cs1], ss0).wait()
    plsc.subcore_barrier()

    def _out_chunk(s, n):
        pltpu.sync_copy(acc.at[pl.ds(s, n)], ra.at[pl.ds(0, n)])
        pltpu.sync_copy(ra.at[pl.ds(0, n)],
                        out_hbm.at[pl.ds(cid * _N + s, n)])

    _over_my_rows(sid, _out_chunk)


def _dinv_of(deg2_ref):
    d = deg2_ref[...]
    deg = d[:, 0] + d[:, 1]
    pos = deg > 0
    return jnp.where(pos, lax.rsqrt(jnp.where(pos, deg, 1.0)), 0.0)


def _lin1_body(deg2_ref, x_ref, w1_ref, h_ref):
    dinv = _dinv_of(deg2_ref)
    h = jnp.dot(x_ref[...], w1_ref[...], preferred_element_type=jnp.float32)
    h_ref[...] = h * dinv[:, None]


def _mid_body(p_ref, deg2_ref, b1_ref, w2_ref, h2_ref):
    dinv = _dinv_of(deg2_ref)
    out1 = (p_ref[0] + p_ref[1]) * dinv[:, None] + b1_ref[...]
    h2 = jnp.maximum(out1, 0.0)
    h2_ref[...] = jnp.dot(
        h2, w2_ref[...], preferred_element_type=jnp.float32) * dinv[:, None]


def _fin_body(q_ref, deg2_ref, b2_ref, o_ref):
    dinv = _dinv_of(deg2_ref)
    o_ref[...] = (q_ref[0] + q_ref[1]) * dinv[:, None] + b2_ref[...]


def kernel(x, edge_index, edge_weight, W1, b1, W2, b2):
    row = edge_index[0].astype(jnp.int32)
    col = edge_index[1].astype(jnp.int32)
    w = edge_weight.astype(jnp.float32)

    deg_raw = _deg_kernel(col, w)
    deg2 = deg_raw.reshape(_NC, _N).T
    b1r = b1.reshape(1, _C)
    b2r = b2.reshape(1, _C)
    grid = (_N // _BLK,)

    h1 = pl.pallas_call(
        _lin1_body,
        grid=grid,
        in_specs=[
            pl.BlockSpec((_BLK, 2), lambda i: (i, 0)),
            pl.BlockSpec((_BLK, _C), lambda i: (i, 0)),
            pl.BlockSpec((_C, _C), lambda i: (0, 0)),
        ],
        out_specs=pl.BlockSpec((_BLK, _C), lambda i: (i, 0)),
        out_shape=jax.ShapeDtypeStruct((_N, _C), jnp.float32),
    )(deg2, x, W1)

    p = _agg_kernel(h1, row, col, w).reshape(_NC, _N, _C)

    h2 = pl.pallas_call(
        _mid_body,
        grid=grid,
        in_specs=[
            pl.BlockSpec((2, _BLK, _C), lambda i: (0, i, 0)),
            pl.BlockSpec((_BLK, 2), lambda i: (i, 0)),
            pl.BlockSpec((1, _C), lambda i: (0, 0)),
            pl.BlockSpec((_C, _C), lambda i: (0, 0)),
        ],
        out_specs=pl.BlockSpec((_BLK, _C), lambda i: (i, 0)),
        out_shape=jax.ShapeDtypeStruct((_N, _C), jnp.float32),
    )(p, deg2, b1r, W2)

    q = _agg_kernel(h2, row, col, w).reshape(_NC, _N, _C)

    out = pl.pallas_call(
        _fin_body,
        grid=grid,
        in_specs=[
            pl.BlockSpec((2, _BLK, _C), lambda i: (0, i, 0)),
            pl.BlockSpec((_BLK, 2), lambda i: (i, 0)),
            pl.BlockSpec((1, _C), lambda i: (0, 0)),
        ],
        out_specs=pl.BlockSpec((_BLK, _C), lambda i: (i, 0)),
        out_shape=jax.ShapeDtypeStruct((_N, _C), jnp.float32),
    )(q, deg2, b2r)
    return out
